# baseline (speedup 1.0000x reference)
import jax
import jax.numpy as jnp
from jax.experimental import pallas as pl
from jax.experimental.pallas import tpu as pltpu

_N = 100000
_F = 128
_NNZ = 2048
_TILE = 4000


def _m_kernel(rows_ref, cols_ref, vals_ref, w_ref, m_ref):
    r = rows_ref[0, :]
    c = cols_ref[0, :]
    v = vals_ref[0, :]
    ids = jax.lax.broadcasted_iota(jnp.int32, (_NNZ, _F), 1)
    r_onehot = (r[:, None] == ids).astype(jnp.float32)
    cv = jnp.where(c[:, None] == ids, v[:, None], 0.0)
    a = jax.lax.dot_general(
        r_onehot, cv, (((0,), (0,)), ((), ())),
        preferred_element_type=jnp.float32)
    m_ref[...] = jax.lax.dot_general(
        w_ref[...], a, (((0,), (0,)), ((), ())),
        preferred_element_type=jnp.float32)


def _mm_kernel(x_ref, m_ref, o_ref):
    o_ref[...] = jnp.dot(x_ref[...], m_ref[...],
                         preferred_element_type=jnp.float32)


def kernel(x, adj_indices, adj_values, W):
    rows = adj_indices[0].reshape(1, _NNZ)
    cols = adj_indices[1].reshape(1, _NNZ)
    vals = adj_values.reshape(1, _NNZ)

    m = pl.pallas_call(
        _m_kernel,
        out_shape=jax.ShapeDtypeStruct((_F, _F), jnp.float32),
    )(rows, cols, vals, W)

    out = pl.pallas_call(
        _mm_kernel,
        grid=(_N // _TILE,),
        in_specs=[
            pl.BlockSpec((_TILE, _F), lambda i: (i, 0)),
            pl.BlockSpec((_F, _F), lambda i: (0, 0)),
        ],
        out_specs=pl.BlockSpec((_TILE, _F), lambda i: (i, 0)),
        out_shape=jax.ShapeDtypeStruct((_N, _F), jnp.float32),
        compiler_params=pltpu.CompilerParams(
            dimension_semantics=("parallel",)),
    )(x, m)
    return out


# speedup vs baseline: 1.0389x; 1.0389x over previous
import jax
import jax.numpy as jnp
from jax.experimental import pallas as pl
from jax.experimental.pallas import tpu as pltpu

_N = 100000
_F = 128
_NNZ = 2048
_TILE = 4000


def _gcn_kernel(rows_ref, cols_ref, vals_ref, w_ref, x_ref, o_ref, m_ref):
    @pl.when(pl.program_id(0) == 0)
    def _build_m():
        r = rows_ref[0, :]
        c = cols_ref[0, :]
        v = vals_ref[0, :]
        ids = jax.lax.broadcasted_iota(jnp.int32, (_NNZ, _F), 1)
        r_onehot = (r[:, None] == ids).astype(jnp.float32)
        cv = jnp.where(c[:, None] == ids, v[:, None], 0.0)
        a = jax.lax.dot_general(
            r_onehot, cv, (((0,), (0,)), ((), ())),
            preferred_element_type=jnp.float32)
        m_ref[...] = jax.lax.dot_general(
            w_ref[...], a, (((0,), (0,)), ((), ())),
            preferred_element_type=jnp.float32)

    o_ref[...] = jnp.dot(x_ref[...], m_ref[...],
                         preferred_element_type=jnp.float32)


def kernel(x, adj_indices, adj_values, W):
    rows = adj_indices[0].reshape(1, _NNZ)
    cols = adj_indices[1].reshape(1, _NNZ)
    vals = adj_values.reshape(1, _NNZ)

    out = pl.pallas_call(
        _gcn_kernel,
        grid=(_N // _TILE,),
        in_specs=[
            pl.BlockSpec((1, _NNZ), lambda i: (0, 0)),
            pl.BlockSpec((1, _NNZ), lambda i: (0, 0)),
            pl.BlockSpec((1, _NNZ), lambda i: (0, 0)),
            pl.BlockSpec((_F, _F), lambda i: (0, 0)),
            pl.BlockSpec((_TILE, _F), lambda i: (i, 0)),
        ],
        out_specs=pl.BlockSpec((_TILE, _F), lambda i: (i, 0)),
        out_shape=jax.ShapeDtypeStruct((_N, _F), jnp.float32),
        scratch_shapes=[pltpu.VMEM((_F, _F), jnp.float32)],
        compiler_params=pltpu.CompilerParams(
            dimension_semantics=("arbitrary",)),
    )(rows, cols, vals, W, x)
    return out


# speedup vs baseline: 1.1897x; 1.1452x over previous
import jax
import jax.numpy as jnp
from jax.experimental import pallas as pl
from jax.experimental.pallas import tpu as pltpu

_N = 100000
_F = 128
_NNZ = 2048
_TILE = 10000


def _gcn_kernel(rows_ref, cols_ref, vals_ref, w_ref, x_ref, o_ref, m_ref):
    @pl.when(pl.program_id(0) == 0)
    def _build_m():
        r = rows_ref[0, :]
        c = cols_ref[0, :]
        v = vals_ref[0, :]
        ids = jax.lax.broadcasted_iota(jnp.int32, (_NNZ, _F), 1)
        r_onehot = (r[:, None] == ids).astype(jnp.float32)
        cv = jnp.where(c[:, None] == ids, v[:, None], 0.0)
        a = jax.lax.dot_general(
            r_onehot, cv, (((0,), (0,)), ((), ())),
            preferred_element_type=jnp.float32)
        m_ref[...] = jax.lax.dot_general(
            w_ref[...], a, (((0,), (0,)), ((), ())),
            preferred_element_type=jnp.float32)

    o_ref[...] = jnp.dot(x_ref[...], m_ref[...],
                         preferred_element_type=jnp.float32)


def kernel(x, adj_indices, adj_values, W):
    rows = adj_indices[0].reshape(1, _NNZ)
    cols = adj_indices[1].reshape(1, _NNZ)
    vals = adj_values.reshape(1, _NNZ)

    out = pl.pallas_call(
        _gcn_kernel,
        grid=(_N // _TILE,),
        in_specs=[
            pl.BlockSpec((1, _NNZ), lambda i: (0, 0)),
            pl.BlockSpec((1, _NNZ), lambda i: (0, 0)),
            pl.BlockSpec((1, _NNZ), lambda i: (0, 0)),
            pl.BlockSpec((_F, _F), lambda i: (0, 0)),
            pl.BlockSpec((_TILE, _F), lambda i: (i, 0)),
        ],
        out_specs=pl.BlockSpec((_TILE, _F), lambda i: (i, 0)),
        out_shape=jax.ShapeDtypeStruct((_N, _F), jnp.float32),
        scratch_shapes=[pltpu.VMEM((_F, _F), jnp.float32)],
        compiler_params=pltpu.CompilerParams(
            dimension_semantics=("arbitrary",)),
    )(rows, cols, vals, W, x)
    return out


# speedup vs baseline: 1.2425x; 1.0444x over previous
import jax
import jax.numpy as jnp
from jax.experimental import pallas as pl
from jax.experimental.pallas import tpu as pltpu

_N = 100000
_F = 128
_NNZ = 2048
_TILE = 20000


def _gcn_kernel(rows_ref, cols_ref, vals_ref, w_ref, x_ref, o_ref, m_ref):
    @pl.when(pl.program_id(0) == 0)
    def _build_m():
        r = rows_ref[0, :]
        c = cols_ref[0, :]
        v = vals_ref[0, :]
        ids = jax.lax.broadcasted_iota(jnp.int32, (_NNZ, _F), 1)
        r_onehot = (r[:, None] == ids).astype(jnp.float32)
        cv = jnp.where(c[:, None] == ids, v[:, None], 0.0)
        a = jax.lax.dot_general(
            r_onehot, cv, (((0,), (0,)), ((), ())),
            preferred_element_type=jnp.float32)
        m_ref[...] = jax.lax.dot_general(
            w_ref[...], a, (((0,), (0,)), ((), ())),
            preferred_element_type=jnp.float32)

    o_ref[...] = jnp.dot(x_ref[...], m_ref[...],
                         preferred_element_type=jnp.float32)


def kernel(x, adj_indices, adj_values, W):
    rows = adj_indices[0].reshape(1, _NNZ)
    cols = adj_indices[1].reshape(1, _NNZ)
    vals = adj_values.reshape(1, _NNZ)

    out = pl.pallas_call(
        _gcn_kernel,
        grid=(_N // _TILE,),
        in_specs=[
            pl.BlockSpec((1, _NNZ), lambda i: (0, 0)),
            pl.BlockSpec((1, _NNZ), lambda i: (0, 0)),
            pl.BlockSpec((1, _NNZ), lambda i: (0, 0)),
            pl.BlockSpec((_F, _F), lambda i: (0, 0)),
            pl.BlockSpec((_TILE, _F), lambda i: (i, 0)),
        ],
        out_specs=pl.BlockSpec((_TILE, _F), lambda i: (i, 0)),
        out_shape=jax.ShapeDtypeStruct((_N, _F), jnp.float32),
        scratch_shapes=[pltpu.VMEM((_F, _F), jnp.float32)],
        compiler_params=pltpu.CompilerParams(
            dimension_semantics=("arbitrary",)),
    )(rows, cols, vals, W, x)
    return out
